# baseline (speedup 1.0000x reference)
import functools

import jax
import jax.numpy as jnp
from jax import lax
from jax.experimental import pallas as pl
from jax.experimental.pallas import tpu as pltpu
from jax.experimental.pallas import tpu_sc as plsc

L = 16
NC, NS = 2, 16
NW = NC * NS

RI = 24
RO = 2 * RI
NK = 9
HT = 16


def _zero_weave_sc(x3, ztile, *, BC, H, W):
    Ho, Wo = 2 * H - 1, 2 * W - 1
    ch_per = BC // NW

    mesh = plsc.VectorSubcoreMesh(
        core_axis_name="c", subcore_axis_name="s", num_cores=NC, num_subcores=NS
    )

    @functools.partial(
        pl.kernel,
        out_type=jax.ShapeDtypeStruct((BC, Ho, Wo), jnp.float32),
        mesh=mesh,
        scratch_types=[
            pltpu.VMEM((RI, W), jnp.float32),
            pltpu.VMEM((RI, W), jnp.float32),
            pltpu.VMEM((HT, W), jnp.float32),
            pltpu.VMEM((HT, W), jnp.float32),
            pltpu.VMEM((RO, Wo), jnp.float32),
            pltpu.VMEM((RO, Wo), jnp.float32),
            pltpu.VMEM((RO, Wo), jnp.float32),
            pltpu.VMEM((HT, Wo), jnp.float32),
            pltpu.VMEM((HT, Wo), jnp.float32),
            pltpu.SemaphoreType.DMA,
            pltpu.SemaphoreType.DMA,
            pltpu.SemaphoreType.DMA,
            pltpu.SemaphoreType.DMA,
            pltpu.SemaphoreType.DMA,
            pltpu.SemaphoreType.DMA,
            pltpu.SemaphoreType.DMA,
            pltpu.SemaphoreType.DMA,
            pltpu.SemaphoreType.DMA,
        ],
        compiler_params=pltpu.CompilerParams(
            use_tc_tiling_on_sc=False, needs_layout_passes=False
        ),
    )
    def zw(x_hbm, z_hbm, out_hbm, ib0, ib1, ihb, itb, obA, obB, obC, ohb, otb,
           si0, si1, sih, sit, soA, soB, soC, soh, sot):
        wid = lax.axis_index("s") * NC + lax.axis_index("c")
        p = wid & 1
        idx16 = wid >> 1

        in_bufs, in_sems = (ib0, ib1), (si0, si1)
        out_bufs, out_sems = (obA, obB, obC), (soA, soB, soC)

        for ob, osem in zip(out_bufs, out_sems):
            pltpu.async_copy(z_hbm, ob, osem)
        pltpu.async_copy(z_hbm.at[pl.ds(0, HT)], ohb, soh)
        pltpu.async_copy(z_hbm.at[pl.ds(0, HT)], otb, sot)

        iota = lax.iota(jnp.int32, L)
        cvecs = [2 * (k * L + iota) for k in range(W // L)]
        nkc = W // L

        def scatter_rows(ib, ob, n_rows, row0_in, roff):
            def do_row(m, c2):
                rvec = lax.broadcast(2 * m + roff, (L,))
                for k in range(nkc):
                    vals = ib[row0_in + m, pl.ds(k * L, L)]
                    plsc.store_scatter(ob, [rvec, cvecs[k]], vals)
                return c2
            lax.fori_loop(0, n_rows, do_row, 0)

        def do_plane(ci, carry):
            ch = 2 * (idx16 * ch_per + ci) + p
            s = ch & 15
            i0 = (s + p) >> 1

            pltpu.async_copy(x_hbm.at[ch, pl.ds(0, HT), :], ihb, sih)
            pltpu.async_copy(x_hbm.at[ch, pl.ds(H - HT, HT), :], itb, sit)
            pltpu.async_copy(x_hbm.at[ch, pl.ds(i0, RI), :], ib0, si0)
            pltpu.async_copy(x_hbm.at[ch, pl.ds(i0 + RI, RI), :], ib1, si1)

            pltpu.make_async_copy(x_hbm.at[ch, pl.ds(0, HT), :], ihb, sih).wait()
            pltpu.make_async_copy(z_hbm.at[pl.ds(0, HT)], ohb, soh).wait()
            scatter_rows(ihb, ohb, HT // 2, 0, 0)
            pltpu.async_copy(ohb, out_hbm.at[ch, pl.ds(0, HT), :], soh)

            for k in range(NK):
                qi, qo = k % 2, k % 3
                pltpu.make_async_copy(
                    x_hbm.at[ch, pl.ds(i0 + k * RI, RI), :],
                    in_bufs[qi], in_sems[qi],
                ).wait()
                pltpu.make_async_copy(z_hbm, out_bufs[qo], out_sems[qo]).wait()
                scatter_rows(in_bufs[qi], out_bufs[qo], RI, 0, p)
                pltpu.async_copy(
                    out_bufs[qo],
                    out_hbm.at[ch, pl.ds(s + k * RO, RO), :],
                    out_sems[qo],
                )
                if k + 2 < NK:
                    pltpu.async_copy(
                        x_hbm.at[ch, pl.ds(i0 + (k + 2) * RI, RI), :],
                        in_bufs[qi], in_sems[qi],
                    )

            pltpu.make_async_copy(x_hbm.at[ch, pl.ds(H - HT, HT), :], itb, sit).wait()
            pltpu.make_async_copy(z_hbm.at[pl.ds(0, HT)], otb, sot).wait()
            scatter_rows(itb, otb, HT // 2, HT // 2, 1)
            pltpu.async_copy(otb, out_hbm.at[ch, pl.ds(Ho - HT, HT), :], sot)
            return carry

        lax.fori_loop(0, ch_per, do_plane, 0)

        for ob, osem in zip(out_bufs, out_sems):
            pltpu.make_async_copy(z_hbm, ob, osem).wait()
        pltpu.make_async_copy(z_hbm.at[pl.ds(0, HT)], ohb, soh).wait()
        pltpu.make_async_copy(z_hbm.at[pl.ds(0, HT)], otb, sot).wait()

    return zw(x3, ztile)


def kernel(x):
    B, C, H, W = x.shape
    Ho, Wo = 2 * H - 1, 2 * W - 1
    x3 = x.reshape(B * C, H, W)
    ztile = jnp.zeros((RO, Wo), jnp.float32)
    out = _zero_weave_sc(x3, ztile, BC=B * C, H=H, W=W)
    return out.reshape(B, C, Ho, Wo)
